# Initial kernel scaffold; baseline (speedup 1.0000x reference)
import jax
import jax.numpy as jnp
from jax.experimental import pallas as pl


def kernel(user, item, rating_mtx, user_similarity, user_bias, item_bias, global_bias):
    raise NotImplementedError("write your pallas kernel here")


# speedup vs baseline: 1.2768x; 1.2768x over previous
import functools

import jax
import jax.numpy as jnp
from jax import lax
from jax.experimental import pallas as pl
from jax.experimental.pallas import tpu as pltpu
from jax.experimental.pallas import tpu_sc as plsc

NU = 4096
NI = 16384
B = 4096

RT = 512
CT = 512


def _k1_body(x_ref, xt_ref, sum_ref, cnt_ref):
    j = pl.program_id(1)
    x = x_ref[...]
    xt_ref[...] = x.T
    mask = x != 0.0
    psum = jnp.sum(jnp.where(mask, x, 0.0), axis=1)
    pcnt = jnp.sum(mask.astype(jnp.float32), axis=1)

    @pl.when(j == 0)
    def _init():
        sum_ref[...] = psum
        cnt_ref[...] = pcnt

    @pl.when(j > 0)
    def _acc():
        sum_ref[...] += psum
        cnt_ref[...] += pcnt


def _transpose_and_rowstats(rating):
    return pl.pallas_call(
        _k1_body,
        grid=(NU // RT, NI // CT),
        in_specs=[pl.BlockSpec((RT, CT), lambda i, j: (i, j))],
        out_specs=[
            pl.BlockSpec((CT, RT), lambda i, j: (j, i)),
            pl.BlockSpec((RT,), lambda i, j: (i,)),
            pl.BlockSpec((RT,), lambda i, j: (i,)),
        ],
        out_shape=[
            jax.ShapeDtypeStruct((NI, NU), jnp.float32),
            jax.ShapeDtypeStruct((NU,), jnp.float32),
            jax.ShapeDtypeStruct((NU,), jnp.float32),
        ],
    )(rating)


_NC = 2
_NS = 16
_NW = _NC * _NS
_BPW = B // _NW
_CH = 16
_NCH = _BPW // _CH


def _k2_body(sim_hbm, rt_hbm, user_hbm, item_hbm, ub_hbm, ib_hbm,
             simg_hbm, colg_hbm, ubg_hbm, ibg_hbm,
             uidx_v, iidx_v, sidx_v, rows_v, brow_v, sem):
    wid = lax.axis_index("s") * _NC + lax.axis_index("c")
    base = wid * _BPW

    pltpu.sync_copy(user_hbm.at[pl.ds(base, _BPW)], uidx_v)
    pltpu.sync_copy(item_hbm.at[pl.ds(base, _BPW)], iidx_v)

    def _gather_bias_rows(idx_v, table, out_hbm):
        for k in range(_BPW // 16):
            sidx_v[pl.ds(k * 16, 16)] = jnp.right_shift(
                idx_v[pl.ds(k * 16, 16)], 7)
        pltpu.async_copy(table.at[sidx_v], brow_v, sem).wait()
        pltpu.sync_copy(brow_v, out_hbm.at[pl.ds(base, _BPW)])

    _gather_bias_rows(uidx_v, ub_hbm, ubg_hbm)
    _gather_bias_rows(iidx_v, ib_hbm, ibg_hbm)

    def _gather_rows(table, idx_v, out_hbm):
        for c in range(_NCH):
            pltpu.async_copy(
                table.at[idx_v.at[pl.ds(c * _CH, _CH)]], rows_v, sem
            ).wait()
            pltpu.sync_copy(
                rows_v, out_hbm.at[pl.ds(base + c * _CH, _CH)]
            )

    _gather_rows(sim_hbm, uidx_v, simg_hbm)
    _gather_rows(rt_hbm, iidx_v, colg_hbm)


def _sc_gather(sim, rating_t, user, item, ub, ib):
    fn = pl.kernel(
        _k2_body,
        mesh=plsc.VectorSubcoreMesh(core_axis_name="c", subcore_axis_name="s"),
        out_type=[
            jax.ShapeDtypeStruct((B, NU), jnp.float32),
            jax.ShapeDtypeStruct((B, NU), jnp.float32),
            jax.ShapeDtypeStruct((B, 128), jnp.float32),
            jax.ShapeDtypeStruct((B, 128), jnp.float32),
        ],
        scratch_types=[
            pltpu.VMEM((_BPW,), jnp.int32),
            pltpu.VMEM((_BPW,), jnp.int32),
            pltpu.VMEM((_BPW,), jnp.int32),
            pltpu.VMEM((_CH, NU), jnp.float32),
            pltpu.VMEM((_BPW, 128), jnp.float32),
            pltpu.SemaphoreType.DMA,
        ],
    )
    return fn(sim, rating_t, user, item,
              ub.reshape(NU // 128, 128), ib.reshape(NI // 128, 128))


BT = 256


def _lane_pick(rows, idx):
    lane = jnp.bitwise_and(idx, 127)
    sel = lax.broadcasted_iota(jnp.int32, rows.shape, 1) == lane[:, None]
    return jnp.sum(jnp.where(sel, rows, 0.0), axis=1)


def _k3_body(simg_ref, colg_ref, sum_ref, cnt_ref, ubg_ref, ibg_ref,
             user_ref, item_ref, gb_ref, out_ref):
    s = sum_ref[...]
    c = cnt_ref[...]
    bias_fixed = jnp.where(c > 0.0, s / jnp.maximum(c, 1.0), 0.0)
    prod = simg_ref[...] * (colg_ref[...] - bias_fixed[None, :])
    acc = jnp.sum(prod, axis=1)
    ubv = _lane_pick(ubg_ref[...], user_ref[...])
    ibv = _lane_pick(ibg_ref[...], item_ref[...])
    score = acc + ubv + ibv + gb_ref[0]
    out_ref[...] = jax.nn.sigmoid(score) * 5.0


def _combine(simg, colg, sums, cnts, ubg, ibg, user, item, gb):
    return pl.pallas_call(
        _k3_body,
        grid=(B // BT,),
        in_specs=[
            pl.BlockSpec((BT, NU), lambda i: (i, 0)),
            pl.BlockSpec((BT, NU), lambda i: (i, 0)),
            pl.BlockSpec((NU,), lambda i: (0,)),
            pl.BlockSpec((NU,), lambda i: (0,)),
            pl.BlockSpec((BT, 128), lambda i: (i, 0)),
            pl.BlockSpec((BT, 128), lambda i: (i, 0)),
            pl.BlockSpec((BT,), lambda i: (i,)),
            pl.BlockSpec((BT,), lambda i: (i,)),
            pl.BlockSpec(memory_space=pltpu.SMEM),
        ],
        out_specs=pl.BlockSpec((BT,), lambda i: (i,)),
        out_shape=jax.ShapeDtypeStruct((B,), jnp.float32),
    )(simg, colg, sums, cnts, ubg, ibg, user, item, gb)


def kernel(user, item, rating_mtx, user_similarity, user_bias, item_bias, global_bias):
    user = user.astype(jnp.int32)
    item = item.astype(jnp.int32)
    rating_t, sums, cnts = _transpose_and_rowstats(rating_mtx)
    simg, colg, ubg, ibg = _sc_gather(
        user_similarity, rating_t, user, item, user_bias, item_bias,
    )
    gb = jnp.reshape(global_bias, (1,)).astype(jnp.float32)
    return _combine(simg, colg, sums, cnts, ubg, ibg, user, item, gb)


# speedup vs baseline: 1.4075x; 1.1023x over previous
import functools

import jax
import jax.numpy as jnp
from jax import lax
from jax.experimental import pallas as pl
from jax.experimental.pallas import tpu as pltpu
from jax.experimental.pallas import tpu_sc as plsc

NU = 4096
NI = 16384
B = 4096

RT = 512
CT = 512


def _k1_body(x_ref, eye_ref, xt_ref, sum_ref, cnt_ref):
    j = pl.program_id(1)
    x = x_ref[...]
    xt = jax.lax.dot_general(
        eye_ref[...], x.astype(jnp.bfloat16),
        (((1,), (1,)), ((), ())),
        preferred_element_type=jnp.float32,
    )
    xt_ref[...] = xt
    mask = xt != 0.0
    psum = jnp.sum(jnp.where(mask, xt, 0.0), axis=0)
    pcnt = jnp.sum(mask.astype(jnp.float32), axis=0)

    @pl.when(j == 0)
    def _init():
        sum_ref[...] = psum
        cnt_ref[...] = pcnt

    @pl.when(j > 0)
    def _acc():
        sum_ref[...] += psum
        cnt_ref[...] += pcnt


def _transpose_and_rowstats(rating):
    return pl.pallas_call(
        _k1_body,
        grid=(NU // RT, NI // CT),
        in_specs=[
            pl.BlockSpec((RT, CT), lambda i, j: (i, j)),
            pl.BlockSpec((RT, CT), lambda i, j: (0, 0)),
        ],
        out_specs=[
            pl.BlockSpec((CT, RT), lambda i, j: (j, i)),
            pl.BlockSpec((RT,), lambda i, j: (i,)),
            pl.BlockSpec((RT,), lambda i, j: (i,)),
        ],
        out_shape=[
            jax.ShapeDtypeStruct((NI, NU), jnp.float32),
            jax.ShapeDtypeStruct((NU,), jnp.float32),
            jax.ShapeDtypeStruct((NU,), jnp.float32),
        ],
    )(rating, jnp.eye(RT, dtype=jnp.bfloat16))


_NC = 2
_NS = 16
_NW = _NC * _NS
_BPW = B // _NW
_CH = 16
_NCH = _BPW // _CH


def _k2_body(sim_hbm, rt_hbm, user_hbm, item_hbm, ub_hbm, ib_hbm,
             simg_hbm, colg_hbm, ubg_hbm, ibg_hbm,
             uidx_v, iidx_v, sidx_v, rows_v, brow_v, sem):
    wid = lax.axis_index("s") * _NC + lax.axis_index("c")
    base = wid * _BPW

    pltpu.sync_copy(user_hbm.at[pl.ds(base, _BPW)], uidx_v)
    pltpu.sync_copy(item_hbm.at[pl.ds(base, _BPW)], iidx_v)

    def _gather_bias_rows(idx_v, table, out_hbm):
        for k in range(_BPW // 16):
            sidx_v[pl.ds(k * 16, 16)] = jnp.right_shift(
                idx_v[pl.ds(k * 16, 16)], 7)
        pltpu.async_copy(table.at[sidx_v], brow_v, sem).wait()
        pltpu.sync_copy(brow_v, out_hbm.at[pl.ds(base, _BPW)])

    _gather_bias_rows(uidx_v, ub_hbm, ubg_hbm)
    _gather_bias_rows(iidx_v, ib_hbm, ibg_hbm)

    def _gather_rows(table, idx_v, out_hbm):
        for c in range(_NCH):
            pltpu.async_copy(
                table.at[idx_v.at[pl.ds(c * _CH, _CH)]], rows_v, sem
            ).wait()
            pltpu.sync_copy(
                rows_v, out_hbm.at[pl.ds(base + c * _CH, _CH)]
            )

    _gather_rows(sim_hbm, uidx_v, simg_hbm)
    _gather_rows(rt_hbm, iidx_v, colg_hbm)


def _sc_gather(sim, rating_t, user, item, ub, ib):
    fn = pl.kernel(
        _k2_body,
        mesh=plsc.VectorSubcoreMesh(core_axis_name="c", subcore_axis_name="s"),
        out_type=[
            jax.ShapeDtypeStruct((B, NU), jnp.float32),
            jax.ShapeDtypeStruct((B, NU), jnp.float32),
            jax.ShapeDtypeStruct((B, 128), jnp.float32),
            jax.ShapeDtypeStruct((B, 128), jnp.float32),
        ],
        scratch_types=[
            pltpu.VMEM((_BPW,), jnp.int32),
            pltpu.VMEM((_BPW,), jnp.int32),
            pltpu.VMEM((_BPW,), jnp.int32),
            pltpu.VMEM((_CH, NU), jnp.float32),
            pltpu.VMEM((_BPW, 128), jnp.float32),
            pltpu.SemaphoreType.DMA,
        ],
    )
    return fn(sim, rating_t, user, item,
              ub.reshape(NU // 128, 128), ib.reshape(NI // 128, 128))


BT = 256


def _lane_pick(rows, idx):
    lane = jnp.bitwise_and(idx, 127)
    sel = lax.broadcasted_iota(jnp.int32, rows.shape, 1) == lane[:, None]
    return jnp.sum(jnp.where(sel, rows, 0.0), axis=1)


def _k3_body(simg_ref, colg_ref, sum_ref, cnt_ref, ubg_ref, ibg_ref,
             user_ref, item_ref, gb_ref, out_ref):
    s = sum_ref[...]
    c = cnt_ref[...]
    bias_fixed = jnp.where(c > 0.0, s / jnp.maximum(c, 1.0), 0.0)
    prod = simg_ref[...] * (colg_ref[...] - bias_fixed[None, :])
    acc = jnp.sum(prod, axis=1)
    ubv = _lane_pick(ubg_ref[...], user_ref[...])
    ibv = _lane_pick(ibg_ref[...], item_ref[...])
    score = acc + ubv + ibv + gb_ref[0]
    out_ref[...] = jax.nn.sigmoid(score) * 5.0


def _combine(simg, colg, sums, cnts, ubg, ibg, user, item, gb):
    return pl.pallas_call(
        _k3_body,
        grid=(B // BT,),
        in_specs=[
            pl.BlockSpec((BT, NU), lambda i: (i, 0)),
            pl.BlockSpec((BT, NU), lambda i: (i, 0)),
            pl.BlockSpec((NU,), lambda i: (0,)),
            pl.BlockSpec((NU,), lambda i: (0,)),
            pl.BlockSpec((BT, 128), lambda i: (i, 0)),
            pl.BlockSpec((BT, 128), lambda i: (i, 0)),
            pl.BlockSpec((BT,), lambda i: (i,)),
            pl.BlockSpec((BT,), lambda i: (i,)),
            pl.BlockSpec(memory_space=pltpu.SMEM),
        ],
        out_specs=pl.BlockSpec((BT,), lambda i: (i,)),
        out_shape=jax.ShapeDtypeStruct((B,), jnp.float32),
    )(simg, colg, sums, cnts, ubg, ibg, user, item, gb)


def kernel(user, item, rating_mtx, user_similarity, user_bias, item_bias, global_bias):
    user = user.astype(jnp.int32)
    item = item.astype(jnp.int32)
    rating_t, sums, cnts = _transpose_and_rowstats(rating_mtx)
    simg, colg, ubg, ibg = _sc_gather(
        user_similarity, rating_t, user, item, user_bias, item_bias,
    )
    gb = jnp.reshape(global_bias, (1,)).astype(jnp.float32)
    return _combine(simg, colg, sums, cnts, ubg, ibg, user, item, gb)


# speedup vs baseline: 1.9432x; 1.3806x over previous
import functools

import jax
import jax.numpy as jnp
from jax import lax
from jax.experimental import pallas as pl
from jax.experimental.pallas import tpu as pltpu
from jax.experimental.pallas import tpu_sc as plsc

NU = 4096
NI = 16384
B = 4096

RT = 512
CT = 512


def _k1_body(x_ref, eye_ref, xt_ref, sum_ref, cnt_ref):
    j = pl.program_id(0)
    xt = jax.lax.dot_general(
        eye_ref[...], x_ref[...].astype(jnp.bfloat16),
        (((1,), (1,)), ((), ())),
        preferred_element_type=jnp.float32,
    )
    xt_ref[...] = xt
    mask = xt != 0.0
    psum = jnp.sum(jnp.where(mask, xt, 0.0), axis=0)
    pcnt = jnp.sum(mask.astype(jnp.float32), axis=0)

    @pl.when(j == 0)
    def _init():
        sum_ref[...] = psum
        cnt_ref[...] = pcnt

    @pl.when(j > 0)
    def _acc():
        sum_ref[...] += psum
        cnt_ref[...] += pcnt


def _transpose_and_rowstats(rating):
    return pl.pallas_call(
        _k1_body,
        grid=(NI // CT,),
        in_specs=[
            pl.BlockSpec((NU, CT), lambda j: (0, j)),
            pl.BlockSpec((CT, CT), lambda j: (0, 0)),
        ],
        out_specs=[
            pl.BlockSpec((CT, NU), lambda j: (j, 0)),
            pl.BlockSpec((NU,), lambda j: (0,)),
            pl.BlockSpec((NU,), lambda j: (0,)),
        ],
        out_shape=[
            jax.ShapeDtypeStruct((NI, NU), jnp.float32),
            jax.ShapeDtypeStruct((NU,), jnp.float32),
            jax.ShapeDtypeStruct((NU,), jnp.float32),
        ],
    )(rating, jnp.eye(CT, dtype=jnp.bfloat16))


_NC = 2
_NS = 16
_NW = _NC * _NS
_BPW = B // _NW
_CH = 8
_NCH = _BPW // _CH


def _gather_rows_pipelined(table, idx_v, out_hbm, base, buf0, buf1,
                           gsems, wsems):
    bufs = (buf0, buf1)
    ghandles = [None] * _NCH
    whandles = [None] * _NCH
    ghandles[0] = pltpu.async_copy(
        table.at[idx_v.at[pl.ds(0, _CH)]], bufs[0], gsems[0])
    for c in range(_NCH):
        if c + 1 < _NCH:
            if c >= 1:
                whandles[c - 1].wait()
            ghandles[c + 1] = pltpu.async_copy(
                table.at[idx_v.at[pl.ds((c + 1) * _CH, _CH)]],
                bufs[(c + 1) % 2], gsems[(c + 1) % 2])
        ghandles[c].wait()
        whandles[c] = pltpu.async_copy(
            bufs[c % 2], out_hbm.at[pl.ds(base + c * _CH, _CH)],
            wsems[c % 2])
    whandles[_NCH - 2].wait()
    whandles[_NCH - 1].wait()


def _k2a_body(sim_hbm, user_hbm, item_hbm, ub_hbm, ib_hbm,
              simg_hbm, ubg_hbm, ibg_hbm,
              uidx_v, iidx_v, sidx_v, rows0_v, rows1_v, brow_v,
              gsem0, gsem1, wsem0, wsem1):
    wid = lax.axis_index("s") * _NC + lax.axis_index("c")
    base = wid * _BPW

    pltpu.sync_copy(user_hbm.at[pl.ds(base, _BPW)], uidx_v)
    pltpu.sync_copy(item_hbm.at[pl.ds(base, _BPW)], iidx_v)

    def _gather_bias_rows(idx_v, table, out_hbm):
        for k in range(_BPW // 16):
            sidx_v[pl.ds(k * 16, 16)] = jnp.right_shift(
                idx_v[pl.ds(k * 16, 16)], 7)
        pltpu.async_copy(table.at[sidx_v], brow_v, gsem0).wait()
        pltpu.sync_copy(brow_v, out_hbm.at[pl.ds(base, _BPW)])

    _gather_bias_rows(uidx_v, ub_hbm, ubg_hbm)
    _gather_bias_rows(iidx_v, ib_hbm, ibg_hbm)

    _gather_rows_pipelined(sim_hbm, uidx_v, simg_hbm, base,
                           rows0_v, rows1_v, (gsem0, gsem1), (wsem0, wsem1))


def _k2b_body(rt_hbm, item_hbm, colg_hbm,
              iidx_v, rows0_v, rows1_v, gsem0, gsem1, wsem0, wsem1):
    wid = lax.axis_index("s") * _NC + lax.axis_index("c")
    base = wid * _BPW
    pltpu.sync_copy(item_hbm.at[pl.ds(base, _BPW)], iidx_v)
    _gather_rows_pipelined(rt_hbm, iidx_v, colg_hbm, base,
                           rows0_v, rows1_v, (gsem0, gsem1), (wsem0, wsem1))


def _sc_gather_sim(sim, user, item, ub, ib):
    fn = pl.kernel(
        _k2a_body,
        mesh=plsc.VectorSubcoreMesh(core_axis_name="c", subcore_axis_name="s"),
        out_type=[
            jax.ShapeDtypeStruct((B, NU), jnp.float32),
            jax.ShapeDtypeStruct((B, 128), jnp.float32),
            jax.ShapeDtypeStruct((B, 128), jnp.float32),
        ],
        scratch_types=[
            pltpu.VMEM((_BPW,), jnp.int32),
            pltpu.VMEM((_BPW,), jnp.int32),
            pltpu.VMEM((_BPW,), jnp.int32),
            pltpu.VMEM((_CH, NU), jnp.float32),
            pltpu.VMEM((_CH, NU), jnp.float32),
            pltpu.VMEM((_BPW, 128), jnp.float32),
            pltpu.SemaphoreType.DMA,
            pltpu.SemaphoreType.DMA,
            pltpu.SemaphoreType.DMA,
            pltpu.SemaphoreType.DMA,
        ],
    )
    return fn(sim, user, item,
              ub.reshape(NU // 128, 128), ib.reshape(NI // 128, 128))


def _sc_gather_cols(rating_t, item):
    fn = pl.kernel(
        _k2b_body,
        mesh=plsc.VectorSubcoreMesh(core_axis_name="c", subcore_axis_name="s"),
        out_type=jax.ShapeDtypeStruct((B, NU), jnp.float32),
        scratch_types=[
            pltpu.VMEM((_BPW,), jnp.int32),
            pltpu.VMEM((_CH, NU), jnp.float32),
            pltpu.VMEM((_CH, NU), jnp.float32),
            pltpu.SemaphoreType.DMA,
            pltpu.SemaphoreType.DMA,
            pltpu.SemaphoreType.DMA,
            pltpu.SemaphoreType.DMA,
        ],
    )
    return fn(rating_t, item)


BT = 256


def _lane_pick(rows, idx):
    lane = jnp.bitwise_and(idx, 127)
    sel = lax.broadcasted_iota(jnp.int32, rows.shape, 1) == lane[:, None]
    return jnp.sum(jnp.where(sel, rows, 0.0), axis=1)


def _k3_body(simg_ref, colg_ref, sum_ref, cnt_ref, ubg_ref, ibg_ref,
             user_ref, item_ref, gb_ref, out_ref):
    s = sum_ref[...]
    c = cnt_ref[...]
    bias_fixed = jnp.where(c > 0.0, s / jnp.maximum(c, 1.0), 0.0)
    prod = simg_ref[...] * (colg_ref[...] - bias_fixed[None, :])
    acc = jnp.sum(prod, axis=1)
    ubv = _lane_pick(ubg_ref[...], user_ref[...])
    ibv = _lane_pick(ibg_ref[...], item_ref[...])
    score = acc + ubv + ibv + gb_ref[0]
    out_ref[...] = jax.nn.sigmoid(score) * 5.0


def _combine(simg, colg, sums, cnts, ubg, ibg, user, item, gb):
    return pl.pallas_call(
        _k3_body,
        grid=(B // BT,),
        in_specs=[
            pl.BlockSpec((BT, NU), lambda i: (i, 0)),
            pl.BlockSpec((BT, NU), lambda i: (i, 0)),
            pl.BlockSpec((NU,), lambda i: (0,)),
            pl.BlockSpec((NU,), lambda i: (0,)),
            pl.BlockSpec((BT, 128), lambda i: (i, 0)),
            pl.BlockSpec((BT, 128), lambda i: (i, 0)),
            pl.BlockSpec((BT,), lambda i: (i,)),
            pl.BlockSpec((BT,), lambda i: (i,)),
            pl.BlockSpec(memory_space=pltpu.SMEM),
        ],
        out_specs=pl.BlockSpec((BT,), lambda i: (i,)),
        out_shape=jax.ShapeDtypeStruct((B,), jnp.float32),
    )(simg, colg, sums, cnts, ubg, ibg, user, item, gb)


def kernel(user, item, rating_mtx, user_similarity, user_bias, item_bias, global_bias):
    user = user.astype(jnp.int32)
    item = item.astype(jnp.int32)
    rating_t, sums, cnts = _transpose_and_rowstats(rating_mtx)
    simg, ubg, ibg = _sc_gather_sim(
        user_similarity, user, item, user_bias, item_bias)
    colg = _sc_gather_cols(rating_t, item)
    gb = jnp.reshape(global_bias, (1,)).astype(jnp.float32)
    return _combine(simg, colg, sums, cnts, ubg, ibg, user, item, gb)


# speedup vs baseline: 2.3935x; 1.2317x over previous
import functools

import jax
import jax.numpy as jnp
from jax import lax
from jax.experimental import pallas as pl
from jax.experimental.pallas import tpu as pltpu
from jax.experimental.pallas import tpu_sc as plsc

NU = 4096
NI = 16384
B = 4096

RT = 512
CT = 512


def _k1_body(x_ref, eye_ref, xt_ref, sum_ref, cnt_ref):
    j = pl.program_id(0)
    xt = jax.lax.dot_general(
        eye_ref[...], x_ref[...].astype(jnp.bfloat16),
        (((1,), (1,)), ((), ())),
        preferred_element_type=jnp.float32,
    )
    bits = jax.lax.bitcast_convert_type(xt, jnp.int32)
    h = NU // 2
    xt_ref[...] = jnp.bitwise_or(
        bits[:, :h],
        jnp.bitwise_and(jnp.right_shift(bits[:, h:], 16), 0xFFFF))
    mask = xt != 0.0
    psum = jnp.sum(jnp.where(mask, xt, 0.0), axis=0)
    pcnt = jnp.sum(mask.astype(jnp.float32), axis=0)

    @pl.when(j == 0)
    def _init():
        sum_ref[...] = psum
        cnt_ref[...] = pcnt

    @pl.when(j > 0)
    def _acc():
        sum_ref[...] += psum
        cnt_ref[...] += pcnt


def _transpose_and_rowstats(rating):
    return pl.pallas_call(
        _k1_body,
        grid=(NI // CT,),
        in_specs=[
            pl.BlockSpec((NU, CT), lambda j: (0, j)),
            pl.BlockSpec((CT, CT), lambda j: (0, 0)),
        ],
        out_specs=[
            pl.BlockSpec((CT, NU // 2), lambda j: (j, 0)),
            pl.BlockSpec((NU,), lambda j: (0,)),
            pl.BlockSpec((NU,), lambda j: (0,)),
        ],
        out_shape=[
            jax.ShapeDtypeStruct((NI, NU // 2), jnp.int32),
            jax.ShapeDtypeStruct((NU,), jnp.float32),
            jax.ShapeDtypeStruct((NU,), jnp.float32),
        ],
    )(rating, jnp.eye(CT, dtype=jnp.bfloat16))


_NC = 2
_NS = 16
_NW = _NC * _NS
_BPW = B // _NW
_CH = 8
_NCH = _BPW // _CH


def _gather_rows_pipelined(table, idx_v, out_hbm, base, buf0, buf1,
                           gsems, wsems):
    bufs = (buf0, buf1)
    ghandles = [None] * _NCH
    whandles = [None] * _NCH
    ghandles[0] = pltpu.async_copy(
        table.at[idx_v.at[pl.ds(0, _CH)]], bufs[0], gsems[0])
    for c in range(_NCH):
        if c + 1 < _NCH:
            if c >= 1:
                whandles[c - 1].wait()
            ghandles[c + 1] = pltpu.async_copy(
                table.at[idx_v.at[pl.ds((c + 1) * _CH, _CH)]],
                bufs[(c + 1) % 2], gsems[(c + 1) % 2])
        ghandles[c].wait()
        whandles[c] = pltpu.async_copy(
            bufs[c % 2], out_hbm.at[pl.ds(base + c * _CH, _CH)],
            wsems[c % 2])
    whandles[_NCH - 2].wait()
    whandles[_NCH - 1].wait()


def _k2a_body(sim_hbm, user_hbm, item_hbm, ub_hbm, ib_hbm,
              simg_hbm, ubg_hbm, ibg_hbm,
              uidx_v, iidx_v, sidx_v, rows0_v, rows1_v, brow_v,
              gsem0, gsem1, wsem0, wsem1):
    wid = lax.axis_index("s") * _NC + lax.axis_index("c")
    base = wid * _BPW

    pltpu.sync_copy(user_hbm.at[pl.ds(base, _BPW)], uidx_v)
    pltpu.sync_copy(item_hbm.at[pl.ds(base, _BPW)], iidx_v)

    def _gather_bias_rows(idx_v, table, out_hbm):
        for k in range(_BPW // 16):
            sidx_v[pl.ds(k * 16, 16)] = jnp.right_shift(
                idx_v[pl.ds(k * 16, 16)], 7)
        pltpu.async_copy(table.at[sidx_v], brow_v, gsem0).wait()
        pltpu.sync_copy(brow_v, out_hbm.at[pl.ds(base, _BPW)])

    _gather_bias_rows(uidx_v, ub_hbm, ubg_hbm)
    _gather_bias_rows(iidx_v, ib_hbm, ibg_hbm)

    _gather_rows_pipelined(sim_hbm, uidx_v, simg_hbm, base,
                           rows0_v, rows1_v, (gsem0, gsem1), (wsem0, wsem1))


def _k2b_body(rt_hbm, item_hbm, colg_hbm,
              iidx_v, rows0_v, rows1_v, gsem0, gsem1, wsem0, wsem1):
    wid = lax.axis_index("s") * _NC + lax.axis_index("c")
    base = wid * _BPW
    pltpu.sync_copy(item_hbm.at[pl.ds(base, _BPW)], iidx_v)
    _gather_rows_pipelined(rt_hbm, iidx_v, colg_hbm, base,
                           rows0_v, rows1_v, (gsem0, gsem1), (wsem0, wsem1))


def _sc_gather_sim(sim, user, item, ub, ib):
    fn = pl.kernel(
        _k2a_body,
        mesh=plsc.VectorSubcoreMesh(core_axis_name="c", subcore_axis_name="s"),
        out_type=[
            jax.ShapeDtypeStruct((B, NU), jnp.float32),
            jax.ShapeDtypeStruct((B, 128), jnp.float32),
            jax.ShapeDtypeStruct((B, 128), jnp.float32),
        ],
        scratch_types=[
            pltpu.VMEM((_BPW,), jnp.int32),
            pltpu.VMEM((_BPW,), jnp.int32),
            pltpu.VMEM((_BPW,), jnp.int32),
            pltpu.VMEM((_CH, NU), jnp.float32),
            pltpu.VMEM((_CH, NU), jnp.float32),
            pltpu.VMEM((_BPW, 128), jnp.float32),
            pltpu.SemaphoreType.DMA,
            pltpu.SemaphoreType.DMA,
            pltpu.SemaphoreType.DMA,
            pltpu.SemaphoreType.DMA,
        ],
    )
    return fn(sim, user, item,
              ub.reshape(NU // 128, 128), ib.reshape(NI // 128, 128))


def _sc_gather_cols(rating_t, item):
    fn = pl.kernel(
        _k2b_body,
        mesh=plsc.VectorSubcoreMesh(core_axis_name="c", subcore_axis_name="s"),
        out_type=jax.ShapeDtypeStruct((B, NU // 2), jnp.int32),
        scratch_types=[
            pltpu.VMEM((_BPW,), jnp.int32),
            pltpu.VMEM((_CH, NU // 2), jnp.int32),
            pltpu.VMEM((_CH, NU // 2), jnp.int32),
            pltpu.SemaphoreType.DMA,
            pltpu.SemaphoreType.DMA,
            pltpu.SemaphoreType.DMA,
            pltpu.SemaphoreType.DMA,
        ],
    )
    return fn(rating_t, item)


BT = 256


def _lane_pick(rows, idx):
    lane = jnp.bitwise_and(idx, 127)
    sel = lax.broadcasted_iota(jnp.int32, rows.shape, 1) == lane[:, None]
    return jnp.sum(jnp.where(sel, rows, 0.0), axis=1)


def _k3_body(simg_ref, colg_ref, sum_ref, cnt_ref, ubg_ref, ibg_ref,
             user_ref, item_ref, gb_ref, out_ref):
    s = sum_ref[...]
    c = cnt_ref[...]
    bias_fixed = jnp.where(c > 0.0, s / jnp.maximum(c, 1.0), 0.0)
    h = NU // 2
    packed = colg_ref[...]
    col_lo = jax.lax.bitcast_convert_type(
        jnp.bitwise_and(packed, jnp.int32(-65536)), jnp.float32)
    col_hi = jax.lax.bitcast_convert_type(
        jnp.left_shift(packed, 16), jnp.float32)
    simg = simg_ref[...]
    prod = (simg[:, :h] * (col_lo - bias_fixed[None, :h])
            + simg[:, h:] * (col_hi - bias_fixed[None, h:]))
    acc = jnp.sum(prod, axis=1)
    ubv = _lane_pick(ubg_ref[...], user_ref[...])
    ibv = _lane_pick(ibg_ref[...], item_ref[...])
    score = acc + ubv + ibv + gb_ref[0]
    out_ref[...] = jax.nn.sigmoid(score) * 5.0


def _combine(simg, colg, sums, cnts, ubg, ibg, user, item, gb):
    return pl.pallas_call(
        _k3_body,
        grid=(B // BT,),
        in_specs=[
            pl.BlockSpec((BT, NU), lambda i: (i, 0)),
            pl.BlockSpec((BT, NU // 2), lambda i: (i, 0)),
            pl.BlockSpec((NU,), lambda i: (0,)),
            pl.BlockSpec((NU,), lambda i: (0,)),
            pl.BlockSpec((BT, 128), lambda i: (i, 0)),
            pl.BlockSpec((BT, 128), lambda i: (i, 0)),
            pl.BlockSpec((BT,), lambda i: (i,)),
            pl.BlockSpec((BT,), lambda i: (i,)),
            pl.BlockSpec(memory_space=pltpu.SMEM),
        ],
        out_specs=pl.BlockSpec((BT,), lambda i: (i,)),
        out_shape=jax.ShapeDtypeStruct((B,), jnp.float32),
    )(simg, colg, sums, cnts, ubg, ibg, user, item, gb)


def kernel(user, item, rating_mtx, user_similarity, user_bias, item_bias, global_bias):
    user = user.astype(jnp.int32)
    item = item.astype(jnp.int32)
    rating_t, sums, cnts = _transpose_and_rowstats(rating_mtx)
    simg, ubg, ibg = _sc_gather_sim(
        user_similarity, user, item, user_bias, item_bias)
    colg = _sc_gather_cols(rating_t, item)
    gb = jnp.reshape(global_bias, (1,)).astype(jnp.float32)
    return _combine(simg, colg, sums, cnts, ubg, ibg, user, item, gb)
